# Initial kernel scaffold; baseline (speedup 1.0000x reference)
import jax
import jax.numpy as jnp
from jax.experimental import pallas as pl


def kernel(loaddata, weeks, years, weekdays, hours, seasonal_delta, cosmic_slope, cosmic_intersection):
    raise NotImplementedError("write your pallas kernel here")


# speedup vs baseline: 133.3926x; 133.3926x over previous
import dataclasses

import jax
import jax.numpy as jnp
from jax import lax
from jax.experimental import pallas as pl
from jax.experimental.pallas import tpu as pltpu
from jax.experimental.pallas import tpu_sc as plsc

_B = 16384
_S = 168
_N = _B * _S
_CHUNK = 4096
_TAB_PAD = 9216

_YEAR0 = 2015


def _sc_lookup(tab_pad, trend_tab, weeks, weekdays, hours, years):
    mesh = plsc.VectorSubcoreMesh(core_axis_name="c", subcore_axis_name="s")
    cp = pltpu.CompilerParams()
    if "needs_layout_passes" in pltpu.CompilerParams.__dataclass_fields__:
        cp = dataclasses.replace(cp, needs_layout_passes=False)

    @pl.kernel(
        compiler_params=cp,
        out_type=jax.ShapeDtypeStruct((_N,), jnp.float32),
        mesh=mesh,
        scratch_types=[
            pltpu.VMEM((_TAB_PAD,), jnp.float32),
            pltpu.VMEM((16,), jnp.float32),
        ],
    )
    def k(tab_hbm, trend_hbm, w_hbm, d_hbm, h_hbm, y_hbm, o_hbm, tab_v, trend_v):
        pltpu.sync_copy(tab_hbm, tab_v)
        pltpu.sync_copy(trend_hbm, trend_v)

        def body(w_ref, d_ref, h_ref, y_ref, o_ref):
            @pl.loop(0, _CHUNK, step=16)
            def _(i):
                s = pl.ds(i, 16)
                w = w_ref[s]
                d = d_ref[s]
                h = h_ref[s]
                y = y_ref[s]
                idx = w * 168 + d * 24 + h
                base = plsc.load_gather(tab_v, [idx])
                trend = plsc.load_gather(trend_v, [y - _YEAR0])
                o_ref[s] = base + trend

        spec = pl.BlockSpec((_CHUNK,), lambda i: (i,))
        pltpu.emit_pipeline(
            body,
            grid=(_N // _CHUNK,),
            in_specs=[spec, spec, spec, spec],
            out_specs=[spec],
            core_axis_name=("c", "s"),
            dimension_semantics=(pltpu.PARALLEL,),
        )(w_hbm, d_hbm, h_hbm, y_hbm, o_hbm)

    return k(tab_pad, trend_tab, weeks, weekdays, hours, years)


@jax.jit
def kernel(loaddata, weeks, years, weekdays, hours, seasonal_delta,
           cosmic_slope, cosmic_intersection):
    del loaddata
    flat = seasonal_delta.reshape(-1)
    tab_pad = jnp.zeros((_TAB_PAD,), jnp.float32)
    tab_pad = lax.dynamic_update_slice(tab_pad, flat, (192,))
    trend_tab = cosmic_intersection + cosmic_slope * jnp.arange(
        16, dtype=jnp.float32)

    out = _sc_lookup(
        tab_pad, trend_tab,
        weeks.reshape(-1), weekdays.reshape(-1),
        hours.reshape(-1), years.reshape(-1),
    )
    return out.reshape(_B, _S, 1)


# speedup vs baseline: 136.5689x; 1.0238x over previous
import dataclasses

import jax
import jax.numpy as jnp
from jax import lax
from jax.experimental import pallas as pl
from jax.experimental.pallas import tpu as pltpu
from jax.experimental.pallas import tpu_sc as plsc

_B = 16384
_S = 168
_N = _B * _S
_COLS = 128
_ROWS = _N // _COLS
_BR = 48
_TAB_PAD = 9216

_YEAR0 = 2015


def _sc_lookup(tab_pad, trend_tab, weeks, weekdays, hours, years):
    mesh = plsc.VectorSubcoreMesh(core_axis_name="c", subcore_axis_name="s")
    cp = pltpu.CompilerParams()
    if "needs_layout_passes" in pltpu.CompilerParams.__dataclass_fields__:
        cp = dataclasses.replace(cp, needs_layout_passes=False)
    cp = dataclasses.replace(cp, use_tc_tiling_on_sc=True)

    @pl.kernel(
        compiler_params=cp,
        out_type=jax.ShapeDtypeStruct((_ROWS, _COLS), jnp.float32),
        mesh=mesh,
        scratch_types=[
            pltpu.VMEM((_TAB_PAD,), jnp.float32),
            pltpu.VMEM((16,), jnp.float32),
        ],
    )
    def k(tab_hbm, trend_hbm, w_hbm, d_hbm, h_hbm, y_hbm, o_hbm, tab_v, trend_v):
        pltpu.sync_copy(tab_hbm, tab_v)
        pltpu.sync_copy(trend_hbm, trend_v)

        def body(w_ref, d_ref, h_ref, y_ref, o_ref):
            @pl.loop(0, _BR)
            def _(r):
                for c in range(0, _COLS, 16):
                    s = pl.ds(c, 16)
                    w = w_ref[r, s]
                    d = d_ref[r, s]
                    h = h_ref[r, s]
                    y = y_ref[r, s]
                    idx = w * 168 + d * 24 + h
                    base = plsc.load_gather(tab_v, [idx])
                    trend = plsc.load_gather(trend_v, [y - _YEAR0])
                    o_ref[r, s] = base + trend

        spec = pl.BlockSpec((_BR, _COLS), lambda i: (i, 0))
        pltpu.emit_pipeline(
            body,
            grid=(_ROWS // _BR,),
            in_specs=[spec, spec, spec, spec],
            out_specs=[spec],
            core_axis_name=("c", "s"),
            dimension_semantics=(pltpu.PARALLEL,),
        )(w_hbm, d_hbm, h_hbm, y_hbm, o_hbm)

    return k(tab_pad, trend_tab, weeks, weekdays, hours, years)


@jax.jit
def kernel(loaddata, weeks, years, weekdays, hours, seasonal_delta,
           cosmic_slope, cosmic_intersection):
    del loaddata
    flat = seasonal_delta.reshape(-1)
    tab_pad = jnp.zeros((_TAB_PAD,), jnp.float32)
    tab_pad = lax.dynamic_update_slice(tab_pad, flat, (192,))
    trend_tab = cosmic_intersection + cosmic_slope * jnp.arange(
        16, dtype=jnp.float32)

    shape2d = (_ROWS, _COLS)
    out = _sc_lookup(
        tab_pad, trend_tab,
        weeks.reshape(shape2d), weekdays.reshape(shape2d),
        hours.reshape(shape2d), years.reshape(shape2d),
    )
    return out.reshape(_B, _S, 1)


# speedup vs baseline: 191.9145x; 1.4053x over previous
import dataclasses

import jax
import jax.numpy as jnp
from jax import lax
from jax.experimental import pallas as pl
from jax.experimental.pallas import tpu as pltpu
from jax.experimental.pallas import tpu_sc as plsc

_B = 16384
_S = 168
_BR = 32
_TAB_PAD = 9216

_YEAR0 = 2015

_COL_STARTS = tuple(range(0, _S - 16, 16)) + (_S - 16,)


def _sc_lookup(tab_pad, trend_tab, weeks, weekdays, hours, years):
    mesh = plsc.VectorSubcoreMesh(core_axis_name="c", subcore_axis_name="s")
    cp = pltpu.CompilerParams()
    if "needs_layout_passes" in pltpu.CompilerParams.__dataclass_fields__:
        cp = dataclasses.replace(cp, needs_layout_passes=False)

    @pl.kernel(
        compiler_params=cp,
        out_type=jax.ShapeDtypeStruct((_B, _S), jnp.float32),
        mesh=mesh,
        scratch_types=[
            pltpu.VMEM((_TAB_PAD,), jnp.float32),
            pltpu.VMEM((16,), jnp.float32),
        ],
    )
    def k(tab_hbm, trend_hbm, w_hbm, d_hbm, h_hbm, y_hbm, o_hbm, tab_v, trend_v):
        pltpu.sync_copy(tab_hbm, tab_v)
        pltpu.sync_copy(trend_hbm, trend_v)

        def body(w_ref, d_ref, h_ref, y_ref, o_ref):
            @pl.loop(0, _BR)
            def _(r):
                for c in _COL_STARTS:
                    s = pl.ds(c, 16)
                    w = w_ref[r, s]
                    d = d_ref[r, s]
                    h = h_ref[r, s]
                    y = y_ref[r, s]
                    idx = w * 168 + d * 24 + h
                    base = plsc.load_gather(tab_v, [idx])
                    trend = plsc.load_gather(trend_v, [y - _YEAR0])
                    o_ref[r, s] = base + trend

        spec = pl.BlockSpec((_BR, _S), lambda i: (i, 0))
        pltpu.emit_pipeline(
            body,
            grid=(_B // _BR,),
            in_specs=[spec, spec, spec, spec],
            out_specs=[spec],
            core_axis_name=("c", "s"),
            dimension_semantics=(pltpu.PARALLEL,),
        )(w_hbm, d_hbm, h_hbm, y_hbm, o_hbm)

    return k(tab_pad, trend_tab, weeks, weekdays, hours, years)


@jax.jit
def kernel(loaddata, weeks, years, weekdays, hours, seasonal_delta,
           cosmic_slope, cosmic_intersection):
    del loaddata
    flat = seasonal_delta.reshape(-1)
    tab_pad = jnp.zeros((_TAB_PAD,), jnp.float32)
    tab_pad = lax.dynamic_update_slice(tab_pad, flat, (192,))
    trend_tab = cosmic_intersection + cosmic_slope * jnp.arange(
        16, dtype=jnp.float32)

    out = _sc_lookup(tab_pad, trend_tab, weeks, weekdays, hours, years)
    return out[..., None]
